# baseline (speedup 1.0000x reference)
import functools

import jax
import jax.numpy as jnp
from jax import lax
from jax.experimental import pallas as pl
from jax.experimental.pallas import tpu as pltpu
from jax.experimental.pallas import tpu_sc as plsc

_N_NODES = 10000
_N_EDGES = 320000
_NC = 2
_NS = 16
_NW = _NC * _NS
_E_W = 9984
_TAIL = _N_EDGES - _NW * _E_W
_CHUNK = 1664
_NCHUNKS = _E_W // _CHUNK
_CVECS = _CHUNK // 16
_AR = 80
_NPAD = _AR * 128


def _sc_partials(u, ei, attr_t):
    mesh = plsc.VectorSubcoreMesh(
        core_axis_name="c", subcore_axis_name="s",
        num_cores=_NC, num_subcores=_NS)

    @functools.partial(
        pl.kernel,
        mesh=mesh,
        out_type=jax.ShapeDtypeStruct((_NC, 3, _AR, 128), jnp.float32),
        compiler_params=pltpu.CompilerParams(needs_layout_passes=False),
        scratch_types=[
            pltpu.VMEM((_N_NODES,), jnp.float32),
            pltpu.VMEM((2, _CHUNK), jnp.int32),
            pltpu.VMEM((2, _CHUNK), jnp.int32),
            pltpu.VMEM((4, _CHUNK), jnp.float32),
            pltpu.VMEM((4, _CHUNK), jnp.float32),
            pltpu.VMEM((_AR, 128), jnp.float32),
            pltpu.VMEM((_AR, 128), jnp.float32),
            pltpu.VMEM((_AR, 128), jnp.float32),
            pltpu.VMEM((1, _AR), jnp.int32),
            pltpu.VMEM_SHARED((_AR, 128), jnp.float32),
            pltpu.VMEM_SHARED((_AR, 128), jnp.float32),
            pltpu.VMEM_SHARED((_AR, 128), jnp.float32),
            pltpu.SemaphoreType.DMA,
            pltpu.SemaphoreType.DMA,
            pltpu.SemaphoreType.DMA,
        ],
    )
    def k(u_hbm, ei_hbm, at_hbm, out_hbm,
          u_v, ei0, ei1, at0, at1, accx, accy, accc, rows_v,
          shx, shy, shc, sem_s, sem_a, sem_b):
        cid = lax.axis_index("c")
        sid = lax.axis_index("s")
        w = cid * _NS + sid
        base = w * _E_W

        ei_bufs = (ei0, ei1)
        at_bufs = (at0, at1)
        sems = (sem_a, sem_b)

        def start_chunk(j, slot):
            st = pl.multiple_of(base + j * _CHUNK, 128)
            de = pltpu.async_copy(
                ei_hbm.at[:, pl.ds(st, _CHUNK)], ei_bufs[slot], sems[slot])
            da = pltpu.async_copy(
                at_hbm.at[:, pl.ds(st, _CHUNK)], at_bufs[slot], sems[slot])
            return de, da

        def wait_chunk(slot):
            pltpu.make_async_copy(
                ei_hbm.at[:, pl.ds(0, _CHUNK)], ei_bufs[slot],
                sems[slot]).wait()
            pltpu.make_async_copy(
                at_hbm.at[:, pl.ds(0, _CHUNK)], at_bufs[slot],
                sems[slot]).wait()

        du_ = pltpu.async_copy(u_hbm, u_v, sem_s)
        start_chunk(0, 0)

        iota16 = lax.iota(jnp.int32, 16)
        for kk in range(_AR // 16):
            rows_v[0, pl.ds(kk * 16, 16)] = iota16 + (kk * 16)

        z16 = jnp.zeros((16,), jnp.float32)

        @plsc.parallel_loop(0, _AR * 8, unroll=8)
        def _(i):
            r = lax.shift_right_logical(i, 3)
            o = lax.bitwise_and(i, 7) * 16
            accx[r, pl.ds(o, 16)] = z16
            accy[r, pl.ds(o, 16)] = z16
            accc[r, pl.ds(o, 16)] = z16

        @pl.when(sid == 0)
        def _():
            pltpu.sync_copy(accx, shx)
            pltpu.sync_copy(accy, shy)
            pltpu.sync_copy(accc, shc)

        plsc.subcore_barrier()
        du_.wait()

        ones_f = jnp.ones((16,), jnp.float32)

        def process(eib, atb, nvecs):
            @plsc.parallel_loop(0, nvecs, unroll=16)
            def _(i):
                off = pl.multiple_of(i * 16, 16)
                ids = eib[0, pl.ds(off, 16)]
                idd = eib[1, pl.ds(off, 16)]
                us = plsc.load_gather(u_v, [ids])
                ud = plsc.load_gather(u_v, [idd])
                du = ud - us
                a0 = atb[0, pl.ds(off, 16)]
                a1 = atb[1, pl.ds(off, 16)]
                r = lax.shift_right_logical(idd, 7)
                c = lax.bitwise_and(idd, 127)
                plsc.addupdate_scatter(accx, [r, c], du / a0)
                plsc.addupdate_scatter(accy, [r, c], du / a1)
                plsc.addupdate_scatter(accc, [r, c], ones_f)

        start_chunk(1, 1)

        @pl.loop(0, _NCHUNKS // 2)
        def _(j):
            for b in range(2):
                cidx = j * 2 + b
                wait_chunk(b)
                process(ei_bufs[b], at_bufs[b], _CVECS)

                @pl.when(cidx + 2 < _NCHUNKS)
                def _():
                    start_chunk(cidx + 2, b)

        @pl.when(w == _NW - 1)
        def _():
            st = _NW * _E_W
            pltpu.sync_copy(ei_hbm.at[:, pl.ds(st, _TAIL)],
                            ei0.at[:, pl.ds(0, _TAIL)])
            pltpu.sync_copy(at_hbm.at[:, pl.ds(st, _TAIL)],
                            at0.at[:, pl.ds(0, _TAIL)])
            process(ei0, at0, _TAIL // 16)

        pltpu.sync_copy(accx, shx.at[rows_v.at[0]], add=True)
        pltpu.sync_copy(accy, shy.at[rows_v.at[0]], add=True)
        pltpu.sync_copy(accc, shc.at[rows_v.at[0]], add=True)

        plsc.subcore_barrier()

        @pl.when(sid == 0)
        def _():
            pltpu.sync_copy(shx, out_hbm.at[cid, 0])
            pltpu.sync_copy(shy, out_hbm.at[cid, 1])
            pltpu.sync_copy(shc, out_hbm.at[cid, 2])

    return k(u, ei, attr_t)


def _combine(parts):
    def ck(p_ref, o_ref):
        p = p_ref[...]
        s = p[0] + p[1]
        num = s[0:2].reshape(2, _NPAD)
        cnt = jnp.maximum(s[2].reshape(1, _NPAD), 1.0)
        o_ref[...] = (num / cnt)[:, :_N_NODES]

    return pl.pallas_call(
        ck,
        out_shape=jax.ShapeDtypeStruct((2, _N_NODES), jnp.float32),
    )(parts)


def kernel(x, edge_index, edge_attr):
    u = x[:, 0]
    parts = _sc_partials(u, edge_index, edge_attr.T)
    o = _combine(parts)
    return o.T


# speedup vs baseline: 1.0458x; 1.0458x over previous
import functools

import jax
import jax.numpy as jnp
from jax import lax
from jax.experimental import pallas as pl
from jax.experimental.pallas import tpu as pltpu
from jax.experimental.pallas import tpu_sc as plsc

_N_NODES = 10000
_N_EDGES = 320000
_NC = 2
_NS = 16
_NW = _NC * _NS
_E_W = 9984
_TAIL = _N_EDGES - _NW * _E_W
_CHUNK = 1664
_NCHUNKS = _E_W // _CHUNK
_CVECS = _CHUNK // 16
_AR = 80
_NPAD = _AR * 128


_UROWS = 80
_UPAD = _UROWS * 128


def _sc_partials(xflat, ei, attr_t):
    mesh = plsc.VectorSubcoreMesh(
        core_axis_name="c", subcore_axis_name="s",
        num_cores=_NC, num_subcores=_NS)

    @functools.partial(
        pl.kernel,
        mesh=mesh,
        out_type=jax.ShapeDtypeStruct((_NC, 3, _AR, 128), jnp.float32),
        compiler_params=pltpu.CompilerParams(needs_layout_passes=False),
        scratch_types=[
            pltpu.VMEM((_UPAD,), jnp.float32),
            pltpu.VMEM((5, 128), jnp.int32),
            pltpu.VMEM_SHARED((_UPAD,), jnp.float32),
            pltpu.VMEM((2, _CHUNK), jnp.int32),
            pltpu.VMEM((2, _CHUNK), jnp.int32),
            pltpu.VMEM((4, _CHUNK), jnp.float32),
            pltpu.VMEM((4, _CHUNK), jnp.float32),
            pltpu.VMEM((_AR, 128), jnp.float32),
            pltpu.VMEM((_AR, 128), jnp.float32),
            pltpu.VMEM((_AR, 128), jnp.float32),
            pltpu.VMEM((1, _AR), jnp.int32),
            pltpu.VMEM_SHARED((_AR, 128), jnp.float32),
            pltpu.VMEM_SHARED((_AR, 128), jnp.float32),
            pltpu.VMEM_SHARED((_AR, 128), jnp.float32),
            pltpu.SemaphoreType.DMA,
            pltpu.SemaphoreType.DMA,
            pltpu.SemaphoreType.DMA,
        ],
    )
    def k(xf_hbm, ei_hbm, at_hbm, out_hbm,
          u_v, uidx_v, u_sh, ei0, ei1, at0, at1, accx, accy, accc, rows_v,
          shx, shy, shc, sem_s, sem_a, sem_b):
        cid = lax.axis_index("c")
        sid = lax.axis_index("s")
        w = cid * _NS + sid
        base = w * _E_W

        iota16 = lax.iota(jnp.int32, 16)

        for m in range(5):
            for j8 in range(8):
                n = (sid * 5 + m) * 128 + j8 * 16 + iota16
                n = jnp.minimum(n, _N_NODES - 1)
                uidx_v[m, pl.ds(j8 * 16, 16)] = n * 128

        ei_bufs = (ei0, ei1)
        at_bufs = (at0, at1)
        sems = (sem_a, sem_b)

        def start_chunk(j, slot):
            st = pl.multiple_of(base + j * _CHUNK, 128)
            de = pltpu.async_copy(
                ei_hbm.at[:, pl.ds(st, _CHUNK)], ei_bufs[slot], sems[slot])
            da = pltpu.async_copy(
                at_hbm.at[:, pl.ds(st, _CHUNK)], at_bufs[slot], sems[slot])
            return de, da

        def wait_chunk(slot):
            pltpu.make_async_copy(
                ei_hbm.at[:, pl.ds(0, _CHUNK)], ei_bufs[slot],
                sems[slot]).wait()
            pltpu.make_async_copy(
                at_hbm.at[:, pl.ds(0, _CHUNK)], at_bufs[slot],
                sems[slot]).wait()

        start_chunk(0, 0)

        ubase = sid * 5 * 128
        ugs = [
            pltpu.async_copy(
                xf_hbm.at[uidx_v.at[m]],
                u_v.at[pl.ds(ubase + m * 128, 128)], sem_s)
            for m in range(5)
        ]

        for kk in range(_AR // 16):
            rows_v[0, pl.ds(kk * 16, 16)] = iota16 + (kk * 16)

        z16 = jnp.zeros((16,), jnp.float32)

        @plsc.parallel_loop(0, _AR * 8, unroll=8)
        def _(i):
            r = lax.shift_right_logical(i, 3)
            o = lax.bitwise_and(i, 7) * 16
            accx[r, pl.ds(o, 16)] = z16
            accy[r, pl.ds(o, 16)] = z16
            accc[r, pl.ds(o, 16)] = z16

        @pl.when(sid == 0)
        def _():
            pltpu.sync_copy(accx, shx)
            pltpu.sync_copy(accy, shy)
            pltpu.sync_copy(accc, shc)

        for d in ugs:
            d.wait()
        pltpu.sync_copy(u_v.at[pl.ds(ubase, 640)], u_sh.at[pl.ds(ubase, 640)])
        plsc.subcore_barrier()
        pltpu.sync_copy(u_sh, u_v)

        ones_f = jnp.ones((16,), jnp.float32)

        def process(eib, atb, nvecs):
            @plsc.parallel_loop(0, nvecs, unroll=8)
            def _(i):
                off = pl.multiple_of(i * 16, 16)
                ids = eib[0, pl.ds(off, 16)]
                idd = eib[1, pl.ds(off, 16)]
                us = plsc.load_gather(u_v, [ids])
                ud = plsc.load_gather(u_v, [idd])
                du = ud - us
                a0 = atb[0, pl.ds(off, 16)]
                a1 = atb[1, pl.ds(off, 16)]
                r = lax.shift_right_logical(idd, 7)
                c = lax.bitwise_and(idd, 127)
                plsc.addupdate_scatter(accx, [r, c], du / a0)
                plsc.addupdate_scatter(accy, [r, c], du / a1)
                plsc.addupdate_scatter(accc, [r, c], ones_f)

        start_chunk(1, 1)

        @pl.loop(0, _NCHUNKS // 2)
        def _(j):
            for b in range(2):
                cidx = j * 2 + b
                wait_chunk(b)
                process(ei_bufs[b], at_bufs[b], _CVECS)

                @pl.when(cidx + 2 < _NCHUNKS)
                def _():
                    start_chunk(cidx + 2, b)

        @pl.when(w == _NW - 1)
        def _():
            st = _NW * _E_W
            pltpu.sync_copy(ei_hbm.at[:, pl.ds(st, _TAIL)],
                            ei0.at[:, pl.ds(0, _TAIL)])
            pltpu.sync_copy(at_hbm.at[:, pl.ds(st, _TAIL)],
                            at0.at[:, pl.ds(0, _TAIL)])
            process(ei0, at0, _TAIL // 16)

        pltpu.sync_copy(accx, shx.at[rows_v.at[0]], add=True)
        pltpu.sync_copy(accy, shy.at[rows_v.at[0]], add=True)
        pltpu.sync_copy(accc, shc.at[rows_v.at[0]], add=True)

        plsc.subcore_barrier()

        @pl.when(sid == 0)
        def _():
            pltpu.sync_copy(shx, out_hbm.at[cid, 0])
            pltpu.sync_copy(shy, out_hbm.at[cid, 1])
            pltpu.sync_copy(shc, out_hbm.at[cid, 2])

    return k(xflat, ei, attr_t)


def _combine(parts):
    def ck(p_ref, o_ref):
        p = p_ref[...]
        s = p[0] + p[1]
        num = s[0:2].reshape(2, _NPAD)
        cnt = jnp.maximum(s[2].reshape(1, _NPAD), 1.0)
        o_ref[...] = (num / cnt)[:, :_N_NODES]

    return pl.pallas_call(
        ck,
        out_shape=jax.ShapeDtypeStruct((2, _N_NODES), jnp.float32),
    )(parts)


def kernel(x, edge_index, edge_attr):
    parts = _sc_partials(x.reshape(-1), edge_index, edge_attr.T)
    o = _combine(parts)
    return o.T
